# baseline (speedup 1.0000x reference)
import functools

import jax
import jax.numpy as jnp
from jax import lax
from jax.experimental import pallas as pl
from jax.experimental.pallas import tpu as pltpu
from jax.experimental.pallas import tpu_sc as plsc

N = 10000
E = 320000
D = 128
H = D // 2
SLOPE = 0.2

NC = 2
NS = 16
C = 128
G2 = 160
NG = G2 // 4
E_PAD = NS * G2 * C
RPT = 632
N_PAD = NS * RPT


def _spmm_body(cr_hbm, vals_hbm, embeds_hbm, out_hbm,
               m0, m1, m2, m3, v0, v1, v2, v3,
               g0, g1, g2, g3, g4, g5, g6, g7,
               i0, i1, i2, i3, i4, i5, i6, i7, acc,
               sm0, sm1, sm2, sm3,
               sg0, sg1, sg2, sg3, sg4, sg5, sg6, sg7,
               ss0, ss1, ss2, ss3, ss4, ss5, ss6, ss7):
    mbufs = (m0, m1, m2, m3)
    vbufs = (v0, v1, v2, v3)
    gbs = (g0, g1, g2, g3, g4, g5, g6, g7)
    ibs = (i0, i1, i2, i3, i4, i5, i6, i7)
    sms = (sm0, sm1, sm2, sm3)
    sgs = (sg0, sg1, sg2, sg3, sg4, sg5, sg6, sg7)
    sss = (ss0, ss1, ss2, ss3, ss4, ss5, ss6, ss7)

    cid = lax.axis_index("c")
    sid = lax.axis_index("s")
    gbase = sid * NG

    def load_meta(mb, j):
        pltpu.async_copy(cr_hbm.at[gbase + j], mbufs[mb], sms[mb])
        pltpu.async_copy(vals_hbm.at[gbase + j], vbufs[mb], sms[mb])

    def wait_meta(mb):
        pltpu.make_async_copy(cr_hbm.at[0], mbufs[mb], sms[mb]).wait()
        pltpu.make_async_copy(vals_hbm.at[0], vbufs[mb], sms[mb]).wait()

    def make_idx(mb, t, b8):
        two = jnp.full((16,), 2, jnp.int32)
        cv = jnp.full((16,), cid, jnp.int32)

        def _q(q, c2):
            w = pl.ds(q * 16, 16)
            ibs[b8][w] = mbufs[mb][2 * t, w] * two + cv
            return c2
        lax.fori_loop(0, C // 16, _q, 0)

    def gather(b8):
        pltpu.async_copy(embeds_hbm.at[ibs[b8]], gbs[b8], sgs[b8])

    def wait_gather(b8):
        pltpu.make_async_copy(embeds_hbm.at[ibs[b8]], gbs[b8],
                              sgs[b8]).wait()

    def scale(mb, t, b8):
        def _grp(q, c2):
            vvec = vbufs[mb][0, pl.ds(t * C + q * 16, 16)]
            for j in range(16):
                s = vvec.at[jnp.full((16,), j, jnp.int32)].get(
                    mode="promise_in_bounds")
                e = q * 16 + j
                for f in range(H // 16):
                    w = pl.ds(f * 16, 16)
                    gbs[b8][e, w] = gbs[b8][e, w] * s
            return c2
        lax.fori_loop(0, C // 16, _grp, 0)

    def scatter(mb, t, b8):
        pltpu.async_copy(gbs[b8], acc.at[mbufs[mb].at[2 * t + 1]], sss[b8],
                         add=True)

    def wait_scatter(mb, t, b8):
        pltpu.make_async_copy(gbs[b8], acc.at[mbufs[mb].at[2 * t + 1]],
                              sss[b8]).wait()

    def chunk(t, b8, mb_prev, mb, mb_next, do_ws, do_gather):
        if do_ws:
            wait_scatter(mb_prev, t, (b8 + 4) % 8)
        if do_gather:
            make_idx(mb_next, t, (b8 + 4) % 8)
            gather((b8 + 4) % 8)
        wait_gather(b8)
        scale(mb, t, b8)
        scatter(mb, t, b8)

    load_meta(0, 0)
    load_meta(1, 1)

    def _zrow(r, carry):
        for f in range(H // 16):
            g0[r, pl.ds(f * 16, 16)] = jnp.zeros((16,), jnp.float32)
        return carry
    lax.fori_loop(0, C, _zrow, 0)
    zbase = sid * RPT
    for k in range(RPT // C):
        pltpu.sync_copy(g0, acc.at[pl.ds(zbase + k * C, C)])
    rem = RPT % C
    pltpu.sync_copy(g0.at[pl.ds(0, rem)],
                    acc.at[pl.ds(zbase + (RPT // C) * C, rem)])
    plsc.subcore_barrier()

    wait_meta(0)
    for t in range(4):
        make_idx(0, t, t)
        gather(t)

    wait_meta(1)
    load_meta(2, 2)
    for t in range(4):
        chunk(t, t, None, 0, 1, do_ws=False, do_gather=True)
    wait_meta(2)
    load_meta(3, 3)
    for t in range(4):
        chunk(t, 4 + t, 0, 1, 2, do_ws=True, do_gather=True)

    def body16(i, carry):
        for m in range(4):
            j = 2 + i * 4 + m
            wait_meta((3 + m) % 4)
            load_meta(m % 4, j + 2)
            for t in range(4):
                b8 = ((m % 2) * 4 + t) % 8
                chunk(t, b8, (1 + m) % 4, (2 + m) % 4, (3 + m) % 4,
                      do_ws=True, do_gather=True)
        return carry
    lax.fori_loop(0, (NG - 4) // 4, body16, 0)

    wait_meta(3)
    for t in range(4):
        chunk(t, t, 1, 2, 3, do_ws=True, do_gather=True)
    for t in range(4):
        chunk(t, 4 + t, 2, 3, None, do_ws=True, do_gather=False)
    for t in range(4):
        wait_scatter(3, t, 4 + t)

    plsc.subcore_barrier()

    obase = sid * RPT
    pltpu.sync_copy(acc.at[pl.ds(obase, RPT)],
                    out_hbm.at[cid, pl.ds(obase, RPT)])


_spmm_sc = functools.partial(
    pl.kernel,
    out_type=jax.ShapeDtypeStruct((NC, N_PAD, H), jnp.float32),
    mesh=plsc.VectorSubcoreMesh(core_axis_name="c", subcore_axis_name="s"),
    compiler_params=pltpu.CompilerParams(use_tc_tiling_on_sc=False),
    scratch_types=(
        [pltpu.VMEM((8, C), jnp.int32) for _ in range(4)]
        + [pltpu.VMEM((1, 4 * C), jnp.float32) for _ in range(4)]
        + [pltpu.VMEM((C, H), jnp.float32) for _ in range(8)]
        + [pltpu.VMEM((C,), jnp.int32) for _ in range(8)]
        + [pltpu.VMEM_SHARED((N_PAD, H), jnp.float32)]
        + [pltpu.SemaphoreType.DMA for _ in range(20)]
    ),
)(_spmm_body)


def _combine_body(p_ref, o_ref):
    o_ref[:, :H] = jnp.where(p_ref[0] > 0, p_ref[0], SLOPE * p_ref[0])
    o_ref[:, H:] = jnp.where(p_ref[1] > 0, p_ref[1], SLOPE * p_ref[1])


def _combine(partials):
    bn = 1000
    return pl.pallas_call(
        _combine_body,
        out_shape=jax.ShapeDtypeStruct((N, D), jnp.float32),
        grid=(N // bn,),
        in_specs=[pl.BlockSpec((NC, bn, H), lambda i: (0, i, 0))],
        out_specs=pl.BlockSpec((bn, D), lambda i: (i, 0)),
    )(partials)


def kernel(adj_indices, adj_values, embeds):
    rows = adj_indices[0].astype(jnp.int32)
    cols = adj_indices[1].astype(jnp.int32)
    vals = adj_values.astype(jnp.float32)
    pad = E_PAD - E
    rows = jnp.pad(rows, (0, pad))
    cols = jnp.pad(cols, (0, pad))
    vals = jnp.pad(vals, (0, pad))
    cr = jnp.stack([cols.reshape(NS * NG, 4, C),
                    rows.reshape(NS * NG, 4, C)],
                   axis=2).reshape(NS * NG, 8, C)
    vg = vals.reshape(NS * NG, 1, 4 * C)
    emb2 = embeds.reshape(2 * N, H)
    partials = _spmm_sc(cr, vg, emb2)
    return _combine(partials)
